# Initial kernel scaffold; baseline (speedup 1.0000x reference)
import jax
import jax.numpy as jnp
from jax.experimental import pallas as pl


def kernel(x, edge_index, edge_weight, Wl0, b0, Wr0, Wl1, b1, Wr1):
    raise NotImplementedError("write your pallas kernel here")


# speedup vs baseline: 5.9411x; 5.9411x over previous
import jax
import jax.numpy as jnp
from jax import lax
from jax.experimental import pallas as pl
from jax.experimental.pallas import tpu as pltpu
from jax.experimental.pallas import tpu_sc as plsc

NC = 2
NS = 16
LANES = 16
KBLK = 128


def _acc_dims(N, NH):
    pb = ((-(-N // NH) + 127) // 128) * 128
    rpt = -(-(pb + 8) // NS)
    rpt = ((rpt + 7) // 8) * 8
    return pb, rpt, rpt * NS


def _edge_agg(table, src, dst, ew, zfeat, ones, *, N, E, T, RW, NH,
              with_count):
    PERTILE = E // NS
    CHUNK = 2000
    NCH = PERTILE // CHUNK
    assert PERTILE % CHUNK == 0 and CHUNK % LANES == 0
    HB, RPT, ACCR = _acc_dims(N, NH)
    CAP = ((PERTILE + KBLK + LANES + KBLK - 1) // KBLK) * KBLK
    DUMMY = HB
    SHIFT = int(HB).bit_length()
    assert (T * N) < (1 << (31 - SHIFT))

    f32 = jnp.float32
    i32 = jnp.int32
    NGRP = CAP // LANES

    def body(table_h, src_h, dst_h, ew_h, zfeat_h, ones_h, a_out, c_out,
             ld_src, ld_dst, ld_ew, com_c, rows, onesb, acc, cntacc, sem):
        cid = lax.axis_index("c")
        sid = lax.axis_index("s")
        pltpu.sync_copy(ones_h, onesb)
        lane = lax.iota(i32, LANES)
        base = sid * PERTILE

        def round_body(r, _):
            t = 2 * cid + lax.div(r, NH)
            h = lax.rem(r, NH)
            lo = h * HB
            pltpu.sync_copy(zfeat_h, acc.at[pl.ds(sid * RPT, RPT)])
            if with_count:
                pltpu.sync_copy(zfeat_h, cntacc.at[pl.ds(sid * RPT, RPT)])
            plsc.subcore_barrier()

            def chunk_body(ci, carry):
                off = pl.multiple_of(base + ci * CHUNK, 8)
                pltpu.sync_copy(src_h.at[pl.ds(off, CHUNK)], ld_src)
                pltpu.sync_copy(dst_h.at[pl.ds(off, CHUNK)], ld_dst)
                pltpu.sync_copy(ew_h.at[pl.ds(off, CHUNK)], ld_ew)

                def cbody(i, carry):
                    ng, cp, pend = carry
                    o = i * LANES
                    ew_v = ld_ew[pl.ds(o, LANES)]
                    d_v = ld_dst[pl.ds(o, LANES)] - lo
                    m = (ew_v == t) & (d_v >= 0) & (d_v < HB)
                    s_v = ld_src[pl.ds(o, LANES)] + t * N
                    mi = m.astype(i32)
                    com = jnp.where(m, lax.shift_left(s_v, SHIFT) + d_v,
                                    DUMMY)
                    _, sv = plsc.sort_key_val(mi, com, descending=True)
                    cnt = jnp.sum(mi)
                    pidx = lax.rem(lane + (LANES - cp), LANES)
                    rot = jnp.take_along_axis(sv, pidx, axis=0)
                    merged = jnp.where(lane >= cp, rot, pend)
                    com_c[pl.ds(ng * LANES, LANES)] = merged
                    total = cp + cnt
                    ov = (total >= LANES).astype(i32)
                    pend = jnp.where(ov > 0, rot, merged)
                    return ng + ov, total - ov * LANES, pend

                return lax.fori_loop(0, CHUNK // LANES, cbody, carry)

            ng, cp, pend = lax.fori_loop(
                0, NCH, chunk_body,
                (jnp.int32(0), jnp.int32(0),
                 jnp.full((LANES,), DUMMY, i32)))

            com_c[pl.ds(ng * LANES, LANES)] = jnp.where(lane < cp, pend,
                                                        DUMMY)

            def blk(g, carry):
                @pl.when(g <= ng)
                def _():
                    v = com_c[pl.ds(g * LANES, LANES)]
                    sidx = lax.shift_right_logical(v, SHIFT)
                    didx = lax.bitwise_and(v, (1 << SHIFT) - 1)
                    pltpu.async_copy(table_h.at[sidx], rows, sem).wait()
                    pltpu.sync_copy(rows, acc.at[didx], add=True)
                    if with_count:
                        pltpu.sync_copy(onesb, cntacc.at[didx], add=True)
                return carry

            lax.fori_loop(0, NGRP, blk, jnp.int32(0))
            plsc.subcore_barrier()

            pltpu.sync_copy(acc.at[pl.ds(sid * RPT, RPT)],
                            a_out.at[t, h, pl.ds(sid * RPT, RPT)])
            if with_count:
                pltpu.sync_copy(cntacc.at[pl.ds(sid * RPT, RPT)],
                                c_out.at[t, h, pl.ds(sid * RPT, RPT)])
            plsc.subcore_barrier()
            return _

        lax.fori_loop(0, 2 * NH, round_body, jnp.int32(0))

    fn = pl.kernel(
        body,
        out_type=(jax.ShapeDtypeStruct((T, NH, ACCR, RW), f32),
                  jax.ShapeDtypeStruct((T, NH, ACCR, RW), f32)),
        mesh=plsc.VectorSubcoreMesh(core_axis_name="c", subcore_axis_name="s",
                                    num_cores=NC, num_subcores=NS),
        compiler_params=pltpu.CompilerParams(needs_layout_passes=False),
        scratch_types=[
            pltpu.VMEM((CHUNK,), i32),
            pltpu.VMEM((CHUNK,), i32),
            pltpu.VMEM((CHUNK,), i32),
            pltpu.VMEM((CAP,), i32),
            pltpu.VMEM((LANES, RW), f32),
            pltpu.VMEM((LANES, RW), f32),
            pltpu.VMEM_SHARED((ACCR, RW), f32),
            pltpu.VMEM_SHARED((ACCR, RW), f32),
            pltpu.SemaphoreType.DMA,
        ],
    )
    return fn(table, src, dst, ew, zfeat, ones)


def _matmul(x, W):
    G, D, H = W.shape
    N = x.shape[0]
    BN = 1000
    NB = N // BN
    assert N % BN == 0

    def mk(x_ref, w_ref, o_ref):
        o_ref[...] = jnp.dot(x_ref[...], w_ref[0],
                             preferred_element_type=jnp.float32)

    return pl.pallas_call(
        mk,
        grid=(NB, G),
        in_specs=[
            pl.BlockSpec((BN, D), lambda nb, g: (nb, 0)),
            pl.BlockSpec((1, D, H), lambda nb, g: (g, 0, 0)),
        ],
        out_specs=pl.BlockSpec((BN, H), lambda nb, g: (g * NB + nb, 0)),
        out_shape=jax.ShapeDtypeStruct((G * N, H), jnp.float32),
    )(x, W)


def _finalize(a_pad, c_pad, z, b, *, N, T, H, RWA, RWC, NHA, NHC):
    PBA, _, _ = _acc_dims(N, NHA)
    PBC, _, _ = _acc_dims(N, NHC)
    BN = 80
    NB = N // BN
    NBA = PBA // BN
    NBC = PBC // BN
    assert N % BN == 0 and PBA % BN == 0 and PBC % BN == 0

    def fk(a_ref, c_ref, z_ref, b_ref, o_ref):
        cnt = jnp.maximum(c_ref[0, 0, :, 0:1], 1.0)
        o_ref[...] = jnp.maximum(
            a_ref[0, 0, :, :H] / cnt + z_ref[...] + b_ref[0], 0.0)

    b3 = b.reshape(T, 1, H)

    return pl.pallas_call(
        fk,
        grid=(T, NB),
        in_specs=[
            pl.BlockSpec((1, 1, BN, RWA), lambda t, nb: (t, nb // NBA,
                                                         nb % NBA, 0)),
            pl.BlockSpec((1, 1, BN, RWC), lambda t, nb: (t, nb // NBC,
                                                         nb % NBC, 0)),
            pl.BlockSpec((BN, H), lambda t, nb: (t * NB + nb, 0)),
            pl.BlockSpec((1, 1, H), lambda t, nb: (t, 0, 0)),
        ],
        out_specs=pl.BlockSpec((BN, H), lambda t, nb: (nb, t)),
        out_shape=jax.ShapeDtypeStruct((N, T * H), jnp.float32),
    )(a_pad, c_pad, z, b3)


def kernel(x, edge_index, edge_weight, Wl0, b0, Wr0, Wl1, b1, Wr1):
    N, D0 = x.shape
    T, _, H = Wl0.shape
    E = edge_weight.shape[0]
    src = edge_index[0]
    dst = edge_index[1]
    ew = edge_weight

    _, RPT0, _ = _acc_dims(N, 4)
    _, RPT1, _ = _acc_dims(N, 2)
    zf0 = jnp.zeros((RPT0, H), jnp.float32)
    zf1 = jnp.zeros((RPT1, H), jnp.float32)
    ones = jnp.ones((LANES, H), jnp.float32)

    yz0 = _matmul(x, jnp.concatenate([Wl0, Wr0], axis=0))
    a0, c0 = _edge_agg(yz0[:T * N], src, dst, ew, zf0, ones,
                       N=N, E=E, T=T, RW=H, NH=4, with_count=True)
    h1 = _finalize(a0, c0, yz0[T * N:], b0, N=N, T=T, H=H,
                   RWA=H, RWC=H, NHA=4, NHC=4)

    yz1 = _matmul(h1, jnp.concatenate([Wl1, Wr1], axis=0))
    a1, _c1 = _edge_agg(yz1[:T * N], src, dst, ew, zf1, ones,
                        N=N, E=E, T=T, RW=H, NH=2, with_count=False)
    return _finalize(a1, c0, yz1[T * N:], b1, N=N, T=T, H=H,
                     RWA=H, RWC=H, NHA=2, NHC=4)
